# Initial kernel scaffold; baseline (speedup 1.0000x reference)
import jax
import jax.numpy as jnp
from jax.experimental import pallas as pl


def kernel(q, mu, Wij, dir_ij, idx_i, idx_j, n_atoms, W1, b1, W2, b2):
    raise NotImplementedError("write your pallas kernel here")


# speedup vs baseline: 13.5460x; 13.5460x over previous
import functools

import jax
import jax.numpy as jnp
from jax import lax
from jax.experimental import pallas as pl
from jax.experimental.pallas import tpu as pltpu
from jax.experimental.pallas import tpu_sc as plsc

N = 10000
E = 320000
F = 128
F3 = 3 * F

NC = 2
NS = 16
NW = NC * NS
PA = 80
NPARTS = N // PA
NPASS = 4
ACC_ROWS = PA + 8
B = 64
L = 16
EBN = 144


def _mlp_body(q_ref, w1_ref, b1_ref, w2_ref, b2_ref, o_ref):
    h = jnp.dot(q_ref[...], w1_ref[...], preferred_element_type=jnp.float32)
    h = h + b1_ref[...]
    h = h * jax.nn.sigmoid(h)
    o_ref[...] = (
        jnp.dot(h, w2_ref[...], preferred_element_type=jnp.float32) + b2_ref[...]
    )


def _mlp(q2, W1, b1, W2, b2):
    blk = 400
    grid = N // blk
    return pl.pallas_call(
        _mlp_body,
        grid=(grid,),
        in_specs=[
            pl.BlockSpec((blk, F), lambda i: (i, 0)),
            pl.BlockSpec((F, F), lambda i: (0, 0)),
            pl.BlockSpec((1, F), lambda i: (0, 0)),
            pl.BlockSpec((F, F3), lambda i: (0, 0)),
            pl.BlockSpec((1, F3), lambda i: (0, 0)),
        ],
        out_specs=pl.BlockSpec((blk, F3), lambda i: (i, 0)),
        out_shape=jax.ShapeDtypeStruct((N, F3), jnp.float32),
    )(q2, W1, b1[None, :], W2, b2[None, :])


def _splat_f32(s):
    return jnp.full((L,), s, dtype=jnp.float32)


def _pick(ref, pos):
    return ref[pl.ds(pos, L)][0]


def _edge_kernel_body(
    x_hbm, q2_hbm, mu2_hbm, wij_hbm, dir_hbm, idxi_hbm, idxj_hbm, eb_hbm,
    qout_hbm, muout_hbm,
    eb_v, idxi_v, idxj_v, dir_v, wij_v, xj_v, muj_v,
    accq, accmu,
):
    c = lax.axis_index("c")
    s = lax.axis_index("s")
    wid = s * NC + c

    pltpu.sync_copy(eb_hbm, eb_v)

    for p_local in range(NPASS):
        pp = p_local * NW + wid
        active = pp < NPARTS
        base_atom = pp * PA

        e_start = _pick(eb_v, pp)
        e_end = _pick(eb_v, pp + 1)
        gs = e_start // B
        ge = (e_end + B - 1) // B
        nblk = jnp.maximum(ge - gs, 0)

        @pl.when(active)
        def _init():
            pltpu.sync_copy(q2_hbm.at[pl.ds(base_atom, PA)],
                            accq.at[pl.ds(0, PA)])
            pltpu.sync_copy(mu2_hbm.at[pl.ds(base_atom, PA)],
                            accmu.at[pl.ds(0, PA)])

        def block_body(k, carry):
            bs = (gs + k) * B
            pltpu.sync_copy(idxi_hbm.at[pl.ds(bs, B)], idxi_v.at[pl.ds(0, B)])
            pltpu.sync_copy(idxj_hbm.at[pl.ds(bs, B)], idxj_v)
            pltpu.sync_copy(dir_hbm.at[pl.ds(bs * 3, 3 * B)],
                            dir_v.at[pl.ds(0, 3 * B)])
            pltpu.sync_copy(wij_hbm.at[pl.ds(bs, B)], wij_v)
            pltpu.sync_copy(x_hbm.at[idxj_v], xj_v)
            pltpu.sync_copy(mu2_hbm.at[idxj_v], muj_v)

            def edge_body(e, carry2):
                ea = bs + e
                inb = (ea >= e_start) & (ea < e_end)
                row_raw = _pick(idxi_v, e) - base_atom
                row = jnp.where(inb, row_raw, PA)
                dv = dir_v[pl.ds(3 * e, L)]
                db = [_splat_f32(dv[d]) for d in range(3)]
                for cc in range(F // L):
                    sl = pl.ds(cc * L, L)
                    slR = pl.ds(F + cc * L, L)
                    slM = pl.ds(2 * F + cc * L, L)
                    accq[row, sl] = accq[row, sl] + wij_v[e, sl] * xj_v[e, sl]
                    a = wij_v[e, slR] * xj_v[e, slR]
                    mm = wij_v[e, slM] * xj_v[e, slM]
                    for d in range(3):
                        sld = pl.ds(d * F + cc * L, L)
                        accmu[row, sld] = (
                            accmu[row, sld] + a * db[d] + mm * muj_v[e, sld]
                        )
                return carry2

            lax.fori_loop(0, B, edge_body, 0)
            return carry

        lax.fori_loop(0, nblk, block_body, 0)

        @pl.when(active)
        def _out():
            pltpu.sync_copy(accq.at[pl.ds(0, PA)],
                            qout_hbm.at[pl.ds(base_atom, PA)])
            pltpu.sync_copy(accmu.at[pl.ds(0, PA)],
                            muout_hbm.at[pl.ds(base_atom, PA)])


def _edge_stage(x, q2, mu2, wij2, dir_flat, idx_i, idx_j, ebounds):
    mesh = plsc.VectorSubcoreMesh(core_axis_name="c", subcore_axis_name="s")
    kern = functools.partial(
        pl.kernel,
        mesh=mesh,
        out_type=[
            jax.ShapeDtypeStruct((N, F), jnp.float32),
            jax.ShapeDtypeStruct((N, F3), jnp.float32),
        ],
        scratch_types=[
            pltpu.VMEM((EBN,), jnp.int32),
            pltpu.VMEM((B + L,), jnp.int32),
            pltpu.VMEM((B,), jnp.int32),
            pltpu.VMEM((3 * B + L,), jnp.float32),
            pltpu.VMEM((B, F3), jnp.float32),
            pltpu.VMEM((B, F3), jnp.float32),
            pltpu.VMEM((B, F3), jnp.float32),
            pltpu.VMEM((ACC_ROWS, F), jnp.float32),
            pltpu.VMEM((ACC_ROWS, F3), jnp.float32),
        ],
    )(_edge_kernel_body)
    return kern(x, q2, mu2, wij2, dir_flat, idx_i, idx_j, ebounds)


def kernel(q, mu, Wij, dir_ij, idx_i, idx_j, n_atoms, W1, b1, W2, b2):
    del n_atoms
    q2 = q.reshape(N, F)
    mu2 = mu.reshape(N, F3)
    wij2 = Wij.reshape(E, F3)

    x = _mlp(q2, W1, b1, W2, b2)

    parts = jnp.arange(NPARTS + 1, dtype=jnp.int32) * PA
    eb = jnp.searchsorted(idx_i, parts, side="left").astype(jnp.int32)
    ebounds = jnp.full((EBN,), E, jnp.int32).at[: NPARTS + 1].set(eb)

    qout, muout = _edge_stage(
        x, q2, mu2, wij2, dir_ij.reshape(E * 3), idx_i.astype(jnp.int32),
        idx_j.astype(jnp.int32), ebounds
    )
    return (qout.reshape(N, 1, F), muout.reshape(N, 3, F))


# speedup vs baseline: 13.8578x; 1.0230x over previous
import functools

import jax
import jax.numpy as jnp
from jax import lax
from jax.experimental import pallas as pl
from jax.experimental.pallas import tpu as pltpu
from jax.experimental.pallas import tpu_sc as plsc

N = 10000
E = 320000
F = 128
F3 = 3 * F

NC = 2
NS = 16
NW = NC * NS
PA = 80
NPARTS = N // PA
NPASS = 4
ACC_ROWS = PA + 8
B = 64
L = 16
GE = 4
EBN = 144


def _mlp_body(q_ref, w1_ref, b1_ref, w2_ref, b2_ref, o_ref):
    h = jnp.dot(q_ref[...], w1_ref[...], preferred_element_type=jnp.float32)
    h = h + b1_ref[...]
    h = h * jax.nn.sigmoid(h)
    o_ref[...] = (
        jnp.dot(h, w2_ref[...], preferred_element_type=jnp.float32) + b2_ref[...]
    )


def _mlp(q2, W1, b1, W2, b2):
    blk = 400
    grid = N // blk
    return pl.pallas_call(
        _mlp_body,
        grid=(grid,),
        in_specs=[
            pl.BlockSpec((blk, F), lambda i: (i, 0)),
            pl.BlockSpec((F, F), lambda i: (0, 0)),
            pl.BlockSpec((1, F), lambda i: (0, 0)),
            pl.BlockSpec((F, F3), lambda i: (0, 0)),
            pl.BlockSpec((1, F3), lambda i: (0, 0)),
        ],
        out_specs=pl.BlockSpec((blk, F3), lambda i: (i, 0)),
        out_shape=jax.ShapeDtypeStruct((N, F3), jnp.float32),
    )(q2, W1, b1[None, :], W2, b2[None, :])


def _splat_f32(s):
    return jnp.full((L,), s, dtype=jnp.float32)


def _pick(ref, pos):
    return ref[pl.ds(pos, L)][0]


def _edge_kernel_body(
    x_hbm, q2_hbm, mu2_hbm, wij_hbm, dir_hbm, idxi_hbm, idxj_hbm, eb_hbm,
    qout_hbm, muout_hbm,
    eb_v, idxi_v, idxj_v, dir_v, wij_v, xj_v, muj_v,
    accq, accmu,
):
    c = lax.axis_index("c")
    s = lax.axis_index("s")
    wid = s * NC + c

    pltpu.sync_copy(eb_hbm, eb_v)

    for p_local in range(NPASS):
        pp = p_local * NW + wid
        active = pp < NPARTS
        base_atom = pp * PA

        e_start = _pick(eb_v, pp)
        e_end = _pick(eb_v, pp + 1)
        gs = e_start // B
        ge = (e_end + B - 1) // B
        nblk = jnp.maximum(ge - gs, 0)

        @pl.when(active)
        def _init():
            pltpu.sync_copy(q2_hbm.at[pl.ds(base_atom, PA)],
                            accq.at[pl.ds(0, PA)])
            pltpu.sync_copy(mu2_hbm.at[pl.ds(base_atom, PA)],
                            accmu.at[pl.ds(0, PA)])

        def block_body(k, carry):
            bs = (gs + k) * B
            pltpu.sync_copy(idxi_hbm.at[pl.ds(bs, B)], idxi_v.at[pl.ds(0, B)])
            pltpu.sync_copy(idxj_hbm.at[pl.ds(bs, B)], idxj_v)
            pltpu.sync_copy(dir_hbm.at[pl.ds(bs * 3, 3 * B)],
                            dir_v.at[pl.ds(0, 3 * B)])
            pltpu.sync_copy(wij_hbm.at[pl.ds(bs, B)], wij_v)
            pltpu.sync_copy(x_hbm.at[idxj_v], xj_v)
            pltpu.sync_copy(mu2_hbm.at[idxj_v], muj_v)

            def group_body(g, carry2):
                e0 = g * GE
                iv16 = idxi_v[pl.ds(e0, L)]
                dv0 = dir_v[pl.ds(3 * e0, L)]
                for u in range(GE):
                    e = e0 + u
                    ea = bs + e
                    inb = (ea >= e_start) & (ea < e_end)
                    row = jnp.where(inb, iv16[u] - base_atom, PA)
                    db = [_splat_f32(dv0[3 * u + d]) for d in range(3)]
                    for cc in range(F // L):
                        sl = pl.ds(cc * L, L)
                        slR = pl.ds(F + cc * L, L)
                        slM = pl.ds(2 * F + cc * L, L)
                        accq[row, sl] = (
                            accq[row, sl] + wij_v[e, sl] * xj_v[e, sl]
                        )
                        a = wij_v[e, slR] * xj_v[e, slR]
                        mm = wij_v[e, slM] * xj_v[e, slM]
                        for d in range(3):
                            sld = pl.ds(d * F + cc * L, L)
                            accmu[row, sld] = (
                                accmu[row, sld] + a * db[d] + mm * muj_v[e, sld]
                            )
                return carry2

            lax.fori_loop(0, B // GE, group_body, 0)
            return carry

        lax.fori_loop(0, nblk, block_body, 0)

        @pl.when(active)
        def _out():
            pltpu.sync_copy(accq.at[pl.ds(0, PA)],
                            qout_hbm.at[pl.ds(base_atom, PA)])
            pltpu.sync_copy(accmu.at[pl.ds(0, PA)],
                            muout_hbm.at[pl.ds(base_atom, PA)])


def _edge_stage(x, q2, mu2, wij2, dir_flat, idx_i, idx_j, ebounds):
    mesh = plsc.VectorSubcoreMesh(core_axis_name="c", subcore_axis_name="s")
    kern = functools.partial(
        pl.kernel,
        mesh=mesh,
        out_type=[
            jax.ShapeDtypeStruct((N, F), jnp.float32),
            jax.ShapeDtypeStruct((N, F3), jnp.float32),
        ],
        scratch_types=[
            pltpu.VMEM((EBN,), jnp.int32),
            pltpu.VMEM((B + L,), jnp.int32),
            pltpu.VMEM((B,), jnp.int32),
            pltpu.VMEM((3 * B + L,), jnp.float32),
            pltpu.VMEM((B, F3), jnp.float32),
            pltpu.VMEM((B, F3), jnp.float32),
            pltpu.VMEM((B, F3), jnp.float32),
            pltpu.VMEM((ACC_ROWS, F), jnp.float32),
            pltpu.VMEM((ACC_ROWS, F3), jnp.float32),
        ],
    )(_edge_kernel_body)
    return kern(x, q2, mu2, wij2, dir_flat, idx_i, idx_j, ebounds)


def kernel(q, mu, Wij, dir_ij, idx_i, idx_j, n_atoms, W1, b1, W2, b2):
    del n_atoms
    q2 = q.reshape(N, F)
    mu2 = mu.reshape(N, F3)
    wij2 = Wij.reshape(E, F3)

    x = _mlp(q2, W1, b1, W2, b2)

    parts = jnp.arange(NPARTS + 1, dtype=jnp.int32) * PA
    eb = jnp.searchsorted(idx_i, parts, side="left").astype(jnp.int32)
    ebounds = jnp.full((EBN,), E, jnp.int32).at[: NPARTS + 1].set(eb)

    qout, muout = _edge_stage(
        x, q2, mu2, wij2, dir_ij.reshape(E * 3), idx_i.astype(jnp.int32),
        idx_j.astype(jnp.int32), ebounds
    )
    return (qout.reshape(N, 1, F), muout.reshape(N, 3, F))
